# baseline (speedup 1.0000x reference)
import jax
import jax.numpy as jnp
from jax import lax
from jax.experimental import pallas as pl
from jax.experimental.pallas import tpu as pltpu

N_ROWS = 64
N_COLS = 32768
BLOCK = 4096
STEPS = N_COLS // BLOCK
SUB = BLOCK // 128
K = 8
CHAIN = SUB // K


def _argmin_body(x_ref, out_ref, rm_scr, ra_scr):
    i = pl.program_id(0)

    @pl.when(i == 0)
    def _():
        rm_scr[...] = jnp.full((N_ROWS, K * 128), jnp.inf, jnp.float32)
        ra_scr[...] = jnp.zeros((N_ROWS, K * 128), jnp.int32)

    for k in range(K):
        rm = rm_scr[:, pl.ds(k * 128, 128)]
        ra = ra_scr[:, pl.ds(k * 128, 128)]
        for c in range(CHAIN):
            s = k * CHAIN + c
            vs = x_ref[:, pl.ds(s * 128, 128)]
            upd = vs < rm
            rm = jnp.where(upd, vs, rm)
            ra = jnp.where(upd, jnp.int32(i * SUB + s), ra)
        rm_scr[:, pl.ds(k * 128, 128)] = rm
        ra_scr[:, pl.ds(k * 128, 128)] = ra

    @pl.when(i == STEPS - 1)
    def _():
        mv = rm_scr[:, pl.ds(0, 128)]
        mi = ra_scr[:, pl.ds(0, 128)]
        for k in range(1, K):
            bv = rm_scr[:, pl.ds(k * 128, 128)]
            bi = ra_scr[:, pl.ds(k * 128, 128)]
            upd = (bv < mv) | ((bv == mv) & (bi < mi))
            mv = jnp.where(upd, bv, mv)
            mi = jnp.where(upd, bi, mi)
        col = mi * 128 + lax.broadcasted_iota(jnp.int32, (N_ROWS, 128), 1)
        m = jnp.min(mv, axis=1, keepdims=True)
        out_ref[...] = jnp.min(
            jnp.where(mv == m, col, jnp.int32(2**30)),
            axis=1, keepdims=True)


_argmin = pl.pallas_call(
    _argmin_body,
    grid=(STEPS,),
    in_specs=[pl.BlockSpec((N_ROWS, BLOCK), lambda i: (0, i))],
    out_specs=pl.BlockSpec((N_ROWS, 1), lambda i: (0, 0)),
    out_shape=jax.ShapeDtypeStruct((N_ROWS, 1), jnp.int32),
    scratch_shapes=[
        pltpu.VMEM((N_ROWS, K * 128), jnp.float32),
        pltpu.VMEM((N_ROWS, K * 128), jnp.int32),
    ],
)


def kernel(x):
    return _argmin(x)[:, 0]
